# Initial kernel scaffold; baseline (speedup 1.0000x reference)
import jax
import jax.numpy as jnp
from jax.experimental import pallas as pl


def kernel(t, beta, alpha):
    raise NotImplementedError("write your pallas kernel here")


# speedup vs baseline: 8.4350x; 8.4350x over previous
import functools

import jax
import jax.numpy as jnp
from jax import lax
from jax.experimental import pallas as pl
from jax.experimental.pallas import tpu as pltpu
from jax.experimental.pallas import tpu_sc as plsc

NUM_TIME_STEPS = 1000
BATCH = 16384
_NC = 2
_NS = 16
_NW = _NC * _NS
_PER = BATCH // _NW
_L = 16


def _sc_gather_kernel(t_hbm, beta_hbm, alpha_hbm, beta_out, alpha_out,
                      idx_v, beta_v, alpha_v, obeta_v, oalpha_v):
    wid = lax.axis_index("s") * _NC + lax.axis_index("c")
    base = wid * _PER

    pltpu.sync_copy(t_hbm.at[pl.ds(base, _PER)], idx_v)
    pltpu.sync_copy(beta_hbm, beta_v)
    pltpu.sync_copy(alpha_hbm, alpha_v)

    def step(i, carry):
        off = i * _L
        idx = idx_v[pl.ds(off, _L)]
        obeta_v[pl.ds(off, _L)] = plsc.load_gather(beta_v, [idx])
        oalpha_v[pl.ds(off, _L)] = plsc.load_gather(alpha_v, [idx])
        return carry

    lax.fori_loop(0, _PER // _L, step, 0)

    pltpu.sync_copy(obeta_v, beta_out.at[pl.ds(base, _PER)])
    pltpu.sync_copy(oalpha_v, alpha_out.at[pl.ds(base, _PER)])


@jax.jit
def kernel(t, beta, alpha):
    mesh = plsc.VectorSubcoreMesh(core_axis_name="c", subcore_axis_name="s")
    out_t = (
        jax.ShapeDtypeStruct((BATCH,), jnp.float32),
        jax.ShapeDtypeStruct((BATCH,), jnp.float32),
    )
    run = functools.partial(
        pl.kernel,
        mesh=mesh,
        out_type=out_t,
        scratch_types=[
            pltpu.VMEM((_PER,), jnp.int32),
            pltpu.VMEM((NUM_TIME_STEPS,), jnp.float32),
            pltpu.VMEM((NUM_TIME_STEPS,), jnp.float32),
            pltpu.VMEM((_PER,), jnp.float32),
            pltpu.VMEM((_PER,), jnp.float32),
        ],
        compiler_params=pltpu.CompilerParams(needs_layout_passes=False),
    )(_sc_gather_kernel)
    return run(t.astype(jnp.int32), beta, alpha)


# speedup vs baseline: 8.8122x; 1.0447x over previous
import functools

import jax
import jax.numpy as jnp
from jax import lax
from jax.experimental import pallas as pl
from jax.experimental.pallas import tpu as pltpu
from jax.experimental.pallas import tpu_sc as plsc

NUM_TIME_STEPS = 1000
BATCH = 16384
_NC = 2
_NS = 16
_NW = _NC * _NS
_PER = BATCH // _NW
_L = 16


def _sc_gather_kernel(t_hbm, beta_hbm, alpha_hbm, beta_out, alpha_out,
                      idx_v, beta_v, alpha_v, obeta_v, oalpha_v, sem):
    wid = lax.axis_index("s") * _NC + lax.axis_index("c")
    base = wid * _PER

    in_copies = [
        pltpu.async_copy(t_hbm.at[pl.ds(base, _PER)], idx_v, sem),
        pltpu.async_copy(beta_hbm, beta_v, sem),
        pltpu.async_copy(alpha_hbm, alpha_v, sem),
    ]
    for c in in_copies:
        c.wait()

    for i in range(_PER // _L):
        off = i * _L
        idx = idx_v[pl.ds(off, _L)]
        obeta_v[pl.ds(off, _L)] = plsc.load_gather(beta_v, [idx])
        oalpha_v[pl.ds(off, _L)] = plsc.load_gather(alpha_v, [idx])

    out_copies = [
        pltpu.async_copy(obeta_v, beta_out.at[pl.ds(base, _PER)], sem),
        pltpu.async_copy(oalpha_v, alpha_out.at[pl.ds(base, _PER)], sem),
    ]
    for c in out_copies:
        c.wait()


@jax.jit
def kernel(t, beta, alpha):
    mesh = plsc.VectorSubcoreMesh(core_axis_name="c", subcore_axis_name="s")
    out_t = (
        jax.ShapeDtypeStruct((BATCH,), jnp.float32),
        jax.ShapeDtypeStruct((BATCH,), jnp.float32),
    )
    run = functools.partial(
        pl.kernel,
        mesh=mesh,
        out_type=out_t,
        scratch_types=[
            pltpu.VMEM((_PER,), jnp.int32),
            pltpu.VMEM((NUM_TIME_STEPS,), jnp.float32),
            pltpu.VMEM((NUM_TIME_STEPS,), jnp.float32),
            pltpu.VMEM((_PER,), jnp.float32),
            pltpu.VMEM((_PER,), jnp.float32),
            pltpu.SemaphoreType.DMA,
        ],
        compiler_params=pltpu.CompilerParams(needs_layout_passes=False),
    )(_sc_gather_kernel)
    return run(t.astype(jnp.int32), beta, alpha)
